# baseline (speedup 1.0000x reference)
import functools

import jax
import jax.numpy as jnp
from jax import lax
from jax.experimental import pallas as pl
from jax.experimental.pallas import tpu as pltpu
from jax.experimental.pallas import tpu_sc as plsc

NN = 100000
NE = 3200000
NB = 8
L = 16
EQ = NE // 4
BLK = 1600
NBLK = EQ // BLK
C = 128
CAP = 1760
NQ = NN // 4

_mesh = plsc.VectorSubcoreMesh(core_axis_name="c", subcore_axis_name="s")


@functools.partial(
    pl.kernel,
    out_type=(jax.ShapeDtypeStruct((NB * NN,), jnp.float32),
              jax.ShapeDtypeStruct((4 * NB * NN,), jnp.float32)),
    mesh=_mesh,
    scratch_types=[
        pltpu.VMEM((NN,), jnp.float32),
        pltpu.VMEM((2 * BLK,), jnp.int32),
        pltpu.VMEM((2 * BLK,), jnp.int32),
        pltpu.VMEM((2 * BLK,), jnp.int32),
        pltpu.VMEM((2 * BLK,), jnp.float32),
        pltpu.VMEM((CAP,), jnp.int32),
        pltpu.VMEM((CAP,), jnp.int32),
        pltpu.VMEM((CAP,), jnp.float32),
        pltpu.VMEM((C,), jnp.float32),
        pltpu.VMEM((BLK,), jnp.float32),
        pltpu.VMEM((4 * BLK,), jnp.float32),
        pltpu.VMEM((L,), jnp.int32),
        pltpu.SemaphoreType.DMA,
        pltpu.SemaphoreType.DMA,
    ],
    compiler_params=pltpu.CompilerParams(needs_layout_passes=False),
)
def _sc_traverse(hflat, heads_h, tails_h, types_h, wts_h, r16_h,
                 out_h, part_h,
                 acc, hb, tb, yb, wb, hix, ctl, cw, hv, ob, cb, rb,
                 sem0, sem1):
    c = lax.axis_index("c")
    s = lax.axis_index("s")
    bb = s // 4
    q = s % 4
    b = c * 4 + bb

    def zero(i, _):
        acc[pl.ds(i * L, L)] = jnp.zeros((L,), jnp.float32)
        return 0
    lax.fori_loop(0, NN // L, zero, 0)

    pltpu.sync_copy(r16_h.at[pl.ds(b * L, L)], rb)
    iot = lax.iota(jnp.int32, L)
    rsp = rb[...]
    bsp = jnp.full((L,), b * NN, jnp.int32)

    def process_chunk(base):
        pltpu.sync_copy(hflat.at[hix.at[pl.ds(base, C)]], hv)
        for j in range(C // L):
            t16 = ctl[pl.ds(base + j * L, L)]
            val = cw[pl.ds(base + j * L, L)] * hv[pl.ds(j * L, L)]
            chk = plsc.load_gather(acc, [t16])

            def rmw(ck):
                need = ck < val
                plsc.store_scatter(acc, [t16], jnp.maximum(ck, val),
                                   mask=need)
                return plsc.load_gather(acc, [t16])

            lax.while_loop(lambda ck: jnp.any(ck < val), rmw, chk)

    srcs = (heads_h, tails_h, types_h, wts_h)
    bufs = (hb, tb, yb, wb)

    def fire(k, boffset, sm):
        off = q * EQ + k * BLK
        for src, buf in zip(srcs, bufs):
            pltpu.async_copy(src.at[pl.ds(off, BLK)],
                             buf.at[pl.ds(boffset, BLK)], sm)

    def drain(sm):
        for src, buf in zip(srcs, bufs):
            pltpu.make_async_copy(src.at[pl.ds(0, BLK)],
                                  buf.at[pl.ds(0, BLK)], sm).wait()

    fire(0, 0, sem0)

    def block(k, fill):
        sl = k % 2
        o2 = sl * BLK

        @pl.when((k + 1 < NBLK) & (sl == 0))
        def _():
            fire(k + 1, BLK, sem1)

        @pl.when((k + 1 < NBLK) & (sl == 1))
        def _():
            fire(k + 1, 0, sem0)

        @pl.when(sl == 0)
        def _():
            drain(sem0)

        @pl.when(sl == 1)
        def _():
            drain(sem1)

        def grp(g, fl):
            base = o2 + g * L
            m = yb[pl.ds(base, L)] == rsp
            cnt = plsc.all_reduce_population_count(m)[0]

            @pl.when(cnt > 0)
            def _():
                csum = plsc.cumsum(m.astype(jnp.int32))
                pos = jnp.full((L,), fl, jnp.int32) + csum - 1
                plsc.store_scatter(hix, [pos], hb[pl.ds(base, L)] + bsp,
                                   mask=m)
                plsc.store_scatter(ctl, [pos], tb[pl.ds(base, L)], mask=m)
                plsc.store_scatter(cw, [pos], wb[pl.ds(base, L)], mask=m)
            return fl + cnt

        fill = lax.fori_loop(0, BLK // L, grp, fill)

        nch = fill // C

        def ch(j, _):
            process_chunk(j * C)
            return 0
        lax.fori_loop(0, nch, ch, 0)

        rem = fill - nch * C

        def mv(i, _):
            so = nch * C + i * L
            v = hix[pl.ds(so, L)]
            hix[pl.ds(i * L, L)] = v
            v = ctl[pl.ds(so, L)]
            ctl[pl.ds(i * L, L)] = v
            v = cw[pl.ds(so, L)]
            cw[pl.ds(i * L, L)] = v
            return 0
        lax.fori_loop(0, (rem + L - 1) // L, mv, 0)
        return rem

    fill = lax.fori_loop(0, NBLK, block, jnp.int32(0))

    fsp = jnp.full((L,), fill, jnp.int32)
    for i in range(C // L):
        lane = jnp.full((L,), i * L, jnp.int32) + iot
        pad = lane >= fsp
        hix[pl.ds(i * L, L)] = jnp.where(pad, bsp, hix[pl.ds(i * L, L)])
        ctl[pl.ds(i * L, L)] = jnp.where(pad, 0, ctl[pl.ds(i * L, L)])
        cw[pl.ds(i * L, L)] = jnp.where(pad, 0.0, cw[pl.ds(i * L, L)])
    process_chunk(0)

    pltpu.sync_copy(acc, part_h.at[pl.ds((b * 4 + q) * NN, NN)])
    plsc.subcore_barrier()

    node0 = q * NQ
    for st, sz in [(i * BLK, BLK) for i in range(15)] + [(15 * BLK, 1000)]:
        for c4 in range(4):
            pltpu.sync_copy(
                part_h.at[pl.ds((b * 4 + c4) * NN + node0 + st, sz)],
                cb.at[pl.ds(c4 * BLK, sz)])

        def cmb(i, _):
            o = i * L
            v = jnp.maximum(
                jnp.maximum(cb[pl.ds(o, L)], cb[pl.ds(BLK + o, L)]),
                jnp.maximum(cb[pl.ds(2 * BLK + o, L)],
                            cb[pl.ds(3 * BLK + o, L)]))
            ob[pl.ds(o, L)] = v
            return 0
        lax.fori_loop(0, sz // L, cmb, 0)
        if sz % L:
            o = sz - L
            v = jnp.maximum(
                jnp.maximum(cb[pl.ds(o, L)], cb[pl.ds(BLK + o, L)]),
                jnp.maximum(cb[pl.ds(2 * BLK + o, L)],
                            cb[pl.ds(3 * BLK + o, L)]))
            ob[pl.ds(o, L)] = v
        pltpu.sync_copy(ob.at[pl.ds(0, sz)],
                        out_h.at[pl.ds(b * NN + node0 + st, sz)])


def kernel(h_prob, edge_weight, edge_index, edge_type, r_index):
    hflat = h_prob.reshape(-1)
    heads = edge_index[0]
    tails = edge_index[1]
    r16 = jnp.repeat(r_index.astype(jnp.int32), L)
    out, _ = _sc_traverse(hflat, heads, tails, edge_type, edge_weight, r16)
    return out.reshape(NB, NN)


# speedup vs baseline: 1.4427x; 1.4427x over previous
import functools

import jax
import jax.numpy as jnp
from jax import lax
from jax.experimental import pallas as pl
from jax.experimental.pallas import tpu as pltpu
from jax.experimental.pallas import tpu_sc as plsc

NN = 100000
NE = 3200000
NB = 8
L = 16
EQ = NE // 4
BLK = 1600
NBLK = EQ // BLK
C = 128
CAP = 1760
NQ = NN // 4

_mesh = plsc.VectorSubcoreMesh(core_axis_name="c", subcore_axis_name="s")


@functools.partial(
    pl.kernel,
    out_type=(jax.ShapeDtypeStruct((NB * NN,), jnp.float32),
              jax.ShapeDtypeStruct((4 * NB * NN,), jnp.float32)),
    mesh=_mesh,
    scratch_types=[
        pltpu.VMEM((NN,), jnp.float32),
        pltpu.VMEM((2 * BLK,), jnp.int32),
        pltpu.VMEM((2 * BLK,), jnp.int32),
        pltpu.VMEM((2 * BLK,), jnp.int32),
        pltpu.VMEM((2 * BLK,), jnp.float32),
        pltpu.VMEM((CAP,), jnp.int32),
        pltpu.VMEM((CAP,), jnp.int32),
        pltpu.VMEM((CAP,), jnp.float32),
        pltpu.VMEM((C,), jnp.float32),
        pltpu.VMEM((BLK,), jnp.float32),
        pltpu.VMEM((4 * BLK,), jnp.float32),
        pltpu.VMEM((L,), jnp.int32),
        pltpu.SemaphoreType.DMA,
        pltpu.SemaphoreType.DMA,
    ],
    compiler_params=pltpu.CompilerParams(needs_layout_passes=False),
)
def _sc_traverse(hflat, heads_h, tails_h, types_h, wts_h, r16_h,
                 out_h, part_h,
                 acc, hb, tb, yb, wb, hix, ctl, cw, hv, ob, cb, rb,
                 sem0, sem1):
    c = lax.axis_index("c")
    s = lax.axis_index("s")
    bb = s // 4
    q = s % 4
    b = c * 4 + bb

    def zero(i, _):
        acc[pl.ds(i * L, L)] = jnp.zeros((L,), jnp.float32)
        return 0
    lax.fori_loop(0, NN // L, zero, 0)

    pltpu.sync_copy(r16_h.at[pl.ds(b * L, L)], rb)
    iot = lax.iota(jnp.int32, L)
    rsp = rb[...]
    bsp = jnp.full((L,), b * NN, jnp.int32)

    def process_chunk(base):
        pltpu.sync_copy(hflat.at[hix.at[pl.ds(base, C)]], hv)
        for j in range(C // L):
            t16 = ctl[pl.ds(base + j * L, L)]
            val = cw[pl.ds(base + j * L, L)] * hv[pl.ds(j * L, L)]
            chk = plsc.load_gather(acc, [t16])

            def rmw(ck):
                need = ck < val
                plsc.store_scatter(acc, [t16], jnp.maximum(ck, val),
                                   mask=need)
                return plsc.load_gather(acc, [t16])

            lax.while_loop(lambda ck: jnp.any(ck < val), rmw, chk)

    srcs = (heads_h, tails_h, types_h, wts_h)
    bufs = (hb, tb, yb, wb)

    def fire(k, boffset, sm):
        off = q * EQ + k * BLK
        for src, buf in zip(srcs, bufs):
            pltpu.async_copy(src.at[pl.ds(off, BLK)],
                             buf.at[pl.ds(boffset, BLK)], sm)

    def drain(sm):
        for src, buf in zip(srcs, bufs):
            pltpu.make_async_copy(src.at[pl.ds(0, BLK)],
                                  buf.at[pl.ds(0, BLK)], sm).wait()

    fire(0, 0, sem0)

    def block(k, fill):
        sl = k % 2
        o2 = sl * BLK

        @pl.when((k + 1 < NBLK) & (sl == 0))
        def _():
            fire(k + 1, BLK, sem1)

        @pl.when((k + 1 < NBLK) & (sl == 1))
        def _():
            fire(k + 1, 0, sem0)

        @pl.when(sl == 0)
        def _():
            drain(sem0)

        @pl.when(sl == 1)
        def _():
            drain(sem1)

        def grp(g, fl):
            base = o2 + g * L
            m = yb[pl.ds(base, L)] == rsp
            csum = plsc.cumsum(m.astype(jnp.int32))
            pos = jnp.full((L,), fl, jnp.int32) + csum - 1
            plsc.store_scatter(hix, [pos], hb[pl.ds(base, L)] + bsp, mask=m)
            plsc.store_scatter(ctl, [pos], tb[pl.ds(base, L)], mask=m)
            plsc.store_scatter(cw, [pos], wb[pl.ds(base, L)], mask=m)
            return fl + csum[L - 1]

        fill = lax.fori_loop(0, BLK // L, grp, fill)

        nch = fill // C

        def ch(j, _):
            process_chunk(j * C)
            return 0
        lax.fori_loop(0, nch, ch, 0)

        rem = fill - nch * C

        def mv(i, _):
            so = nch * C + i * L
            v = hix[pl.ds(so, L)]
            hix[pl.ds(i * L, L)] = v
            v = ctl[pl.ds(so, L)]
            ctl[pl.ds(i * L, L)] = v
            v = cw[pl.ds(so, L)]
            cw[pl.ds(i * L, L)] = v
            return 0
        lax.fori_loop(0, (rem + L - 1) // L, mv, 0)
        return rem

    fill = lax.fori_loop(0, NBLK, block, jnp.int32(0))

    fsp = jnp.full((L,), fill, jnp.int32)
    for i in range(C // L):
        lane = jnp.full((L,), i * L, jnp.int32) + iot
        pad = lane >= fsp
        hix[pl.ds(i * L, L)] = jnp.where(pad, bsp, hix[pl.ds(i * L, L)])
        ctl[pl.ds(i * L, L)] = jnp.where(pad, 0, ctl[pl.ds(i * L, L)])
        cw[pl.ds(i * L, L)] = jnp.where(pad, 0.0, cw[pl.ds(i * L, L)])
    process_chunk(0)

    pltpu.sync_copy(acc, part_h.at[pl.ds((b * 4 + q) * NN, NN)])
    plsc.subcore_barrier()

    node0 = q * NQ
    for st, sz in [(i * BLK, BLK) for i in range(15)] + [(15 * BLK, 1000)]:
        for c4 in range(4):
            pltpu.sync_copy(
                part_h.at[pl.ds((b * 4 + c4) * NN + node0 + st, sz)],
                cb.at[pl.ds(c4 * BLK, sz)])

        def cmb(i, _):
            o = i * L
            v = jnp.maximum(
                jnp.maximum(cb[pl.ds(o, L)], cb[pl.ds(BLK + o, L)]),
                jnp.maximum(cb[pl.ds(2 * BLK + o, L)],
                            cb[pl.ds(3 * BLK + o, L)]))
            ob[pl.ds(o, L)] = v
            return 0
        lax.fori_loop(0, sz // L, cmb, 0)
        if sz % L:
            o = sz - L
            v = jnp.maximum(
                jnp.maximum(cb[pl.ds(o, L)], cb[pl.ds(BLK + o, L)]),
                jnp.maximum(cb[pl.ds(2 * BLK + o, L)],
                            cb[pl.ds(3 * BLK + o, L)]))
            ob[pl.ds(o, L)] = v
        pltpu.sync_copy(ob.at[pl.ds(0, sz)],
                        out_h.at[pl.ds(b * NN + node0 + st, sz)])


def kernel(h_prob, edge_weight, edge_index, edge_type, r_index):
    hflat = h_prob.reshape(-1)
    heads = edge_index[0]
    tails = edge_index[1]
    r16 = jnp.repeat(r_index.astype(jnp.int32), L)
    out, _ = _sc_traverse(hflat, heads, tails, edge_type, edge_weight, r16)
    return out.reshape(NB, NN)


# speedup vs baseline: 1.5122x; 1.0482x over previous
import functools

import jax
import jax.numpy as jnp
from jax import lax
from jax.experimental import pallas as pl
from jax.experimental.pallas import tpu as pltpu
from jax.experimental.pallas import tpu_sc as plsc

NN = 100000
NE = 3200000
NB = 8
L = 16
EQ = NE // 4
BLK = 1600
NBLK = EQ // BLK
C = 128
CAP = 1760
NQ = NN // 4

_mesh = plsc.VectorSubcoreMesh(core_axis_name="c", subcore_axis_name="s")


@functools.partial(
    pl.kernel,
    out_type=(jax.ShapeDtypeStruct((NB * NN,), jnp.float32),
              jax.ShapeDtypeStruct((4 * NB * NN,), jnp.float32)),
    mesh=_mesh,
    scratch_types=[
        pltpu.VMEM((NN,), jnp.float32),
        pltpu.VMEM((2 * BLK,), jnp.int32),
        pltpu.VMEM((2 * BLK,), jnp.int32),
        pltpu.VMEM((2 * BLK,), jnp.int32),
        pltpu.VMEM((2 * BLK,), jnp.float32),
        pltpu.VMEM((CAP,), jnp.int32),
        pltpu.VMEM((CAP,), jnp.int32),
        pltpu.VMEM((CAP,), jnp.float32),
        pltpu.VMEM((C,), jnp.float32),
        pltpu.VMEM((BLK,), jnp.float32),
        pltpu.VMEM((4 * BLK,), jnp.float32),
        pltpu.VMEM((L,), jnp.int32),
        pltpu.SemaphoreType.DMA,
        pltpu.SemaphoreType.DMA,
    ],
    compiler_params=pltpu.CompilerParams(needs_layout_passes=False),
)
def _sc_traverse(hflat, heads_h, tails_h, types_h, wts_h, r16_h,
                 out_h, part_h,
                 acc, hb, tb, yb, wb, hix, ctl, cw, hv, ob, cb, rb,
                 sem0, sem1):
    c = lax.axis_index("c")
    s = lax.axis_index("s")
    bb = s // 4
    q = s % 4
    b = c * 4 + bb

    def zero(i, _):
        acc[pl.ds(i * L, L)] = jnp.zeros((L,), jnp.float32)
        return 0
    lax.fori_loop(0, NN // L, zero, 0)

    pltpu.sync_copy(r16_h.at[pl.ds(b * L, L)], rb)
    iot = lax.iota(jnp.int32, L)
    rsp = rb[...]
    bsp = jnp.full((L,), b * NN, jnp.int32)

    def process_chunk(base):
        pltpu.sync_copy(hflat.at[hix.at[pl.ds(base, C)]], hv)
        for j in range(C // L):
            t16 = ctl[pl.ds(base + j * L, L)]
            val = cw[pl.ds(base + j * L, L)] * hv[pl.ds(j * L, L)]
            chk = plsc.load_gather(acc, [t16])

            def rmw(ck):
                need = ck < val
                plsc.store_scatter(acc, [t16], jnp.maximum(ck, val),
                                   mask=need)
                return plsc.load_gather(acc, [t16])

            lax.while_loop(lambda ck: jnp.any(ck < val), rmw, chk)

    srcs = (heads_h, tails_h, types_h, wts_h)
    bufs = (hb, tb, yb, wb)

    def fire(k, boffset, sm):
        off = q * EQ + k * BLK
        for src, buf in zip(srcs, bufs):
            pltpu.async_copy(src.at[pl.ds(off, BLK)],
                             buf.at[pl.ds(boffset, BLK)], sm)

    def drain(sm):
        for src, buf in zip(srcs, bufs):
            pltpu.make_async_copy(src.at[pl.ds(0, BLK)],
                                  buf.at[pl.ds(0, BLK)], sm).wait()

    fire(0, 0, sem0)

    def block(k, fill):
        sl = k % 2
        o2 = sl * BLK

        @pl.when((k + 1 < NBLK) & (sl == 0))
        def _():
            fire(k + 1, BLK, sem1)

        @pl.when((k + 1 < NBLK) & (sl == 1))
        def _():
            fire(k + 1, 0, sem0)

        @pl.when(sl == 0)
        def _():
            drain(sem0)

        @pl.when(sl == 1)
        def _():
            drain(sem1)

        def grp(g, fl):
            base = o2 + g * L
            m = yb[pl.ds(base, L)] == rsp
            csum = plsc.cumsum(m.astype(jnp.int32))
            pos = jnp.full((L,), fl, jnp.int32) + csum - 1
            plsc.store_scatter(hix, [pos], hb[pl.ds(base, L)] + bsp, mask=m)
            plsc.store_scatter(ctl, [pos], tb[pl.ds(base, L)], mask=m)
            plsc.store_scatter(cw, [pos], wb[pl.ds(base, L)], mask=m)
            return fl + plsc.all_reduce_population_count(m)[0]

        fill = lax.fori_loop(0, BLK // L, grp, fill)

        nch = fill // C

        def ch(j, _):
            process_chunk(j * C)
            return 0
        lax.fori_loop(0, nch, ch, 0)

        rem = fill - nch * C

        def mv(i, _):
            so = nch * C + i * L
            v = hix[pl.ds(so, L)]
            hix[pl.ds(i * L, L)] = v
            v = ctl[pl.ds(so, L)]
            ctl[pl.ds(i * L, L)] = v
            v = cw[pl.ds(so, L)]
            cw[pl.ds(i * L, L)] = v
            return 0
        lax.fori_loop(0, (rem + L - 1) // L, mv, 0)
        return rem

    fill = lax.fori_loop(0, NBLK, block, jnp.int32(0))

    fsp = jnp.full((L,), fill, jnp.int32)
    for i in range(C // L):
        lane = jnp.full((L,), i * L, jnp.int32) + iot
        pad = lane >= fsp
        hix[pl.ds(i * L, L)] = jnp.where(pad, bsp, hix[pl.ds(i * L, L)])
        ctl[pl.ds(i * L, L)] = jnp.where(pad, 0, ctl[pl.ds(i * L, L)])
        cw[pl.ds(i * L, L)] = jnp.where(pad, 0.0, cw[pl.ds(i * L, L)])
    process_chunk(0)

    pltpu.sync_copy(acc, part_h.at[pl.ds((b * 4 + q) * NN, NN)])
    plsc.subcore_barrier()

    node0 = q * NQ
    for st, sz in [(i * BLK, BLK) for i in range(15)] + [(15 * BLK, 1000)]:
        for c4 in range(4):
            pltpu.sync_copy(
                part_h.at[pl.ds((b * 4 + c4) * NN + node0 + st, sz)],
                cb.at[pl.ds(c4 * BLK, sz)])

        def cmb(i, _):
            o = i * L
            v = jnp.maximum(
                jnp.maximum(cb[pl.ds(o, L)], cb[pl.ds(BLK + o, L)]),
                jnp.maximum(cb[pl.ds(2 * BLK + o, L)],
                            cb[pl.ds(3 * BLK + o, L)]))
            ob[pl.ds(o, L)] = v
            return 0
        lax.fori_loop(0, sz // L, cmb, 0)
        if sz % L:
            o = sz - L
            v = jnp.maximum(
                jnp.maximum(cb[pl.ds(o, L)], cb[pl.ds(BLK + o, L)]),
                jnp.maximum(cb[pl.ds(2 * BLK + o, L)],
                            cb[pl.ds(3 * BLK + o, L)]))
            ob[pl.ds(o, L)] = v
        pltpu.sync_copy(ob.at[pl.ds(0, sz)],
                        out_h.at[pl.ds(b * NN + node0 + st, sz)])


def kernel(h_prob, edge_weight, edge_index, edge_type, r_index):
    hflat = h_prob.reshape(-1)
    heads = edge_index[0]
    tails = edge_index[1]
    r16 = jnp.repeat(r_index.astype(jnp.int32), L)
    out, _ = _sc_traverse(hflat, heads, tails, edge_type, edge_weight, r16)
    return out.reshape(NB, NN)


# speedup vs baseline: 1.5427x; 1.0202x over previous
import functools

import jax
import jax.numpy as jnp
from jax import lax
from jax.experimental import pallas as pl
from jax.experimental.pallas import tpu as pltpu
from jax.experimental.pallas import tpu_sc as plsc

NN = 100000
NE = 3200000
NB = 8
L = 16
EQ = NE // 4
BLK = 1600
NBLK = EQ // BLK
C = 128
CAP = 1760
NQ = NN // 4

_mesh = plsc.VectorSubcoreMesh(core_axis_name="c", subcore_axis_name="s")


@functools.partial(
    pl.kernel,
    out_type=(jax.ShapeDtypeStruct((NB * NN,), jnp.float32),
              jax.ShapeDtypeStruct((4 * NB * NN,), jnp.float32)),
    mesh=_mesh,
    scratch_types=[
        pltpu.VMEM((NN,), jnp.float32),
        pltpu.VMEM((2 * BLK,), jnp.int32),
        pltpu.VMEM((2 * BLK,), jnp.int32),
        pltpu.VMEM((2 * BLK,), jnp.int32),
        pltpu.VMEM((2 * BLK,), jnp.float32),
        pltpu.VMEM((CAP,), jnp.int32),
        pltpu.VMEM((CAP,), jnp.int32),
        pltpu.VMEM((CAP,), jnp.float32),
        pltpu.VMEM((C,), jnp.float32),
        pltpu.VMEM((BLK,), jnp.float32),
        pltpu.VMEM((4 * BLK,), jnp.float32),
        pltpu.VMEM((L,), jnp.int32),
        pltpu.SemaphoreType.DMA,
        pltpu.SemaphoreType.DMA,
    ],
    compiler_params=pltpu.CompilerParams(needs_layout_passes=False),
)
def _sc_traverse(hflat, heads_h, tails_h, types_h, wts_h, r16_h,
                 out_h, part_h,
                 acc, hb, tb, yb, wb, hix, ctl, cw, hv, ob, cb, rb,
                 sem0, sem1):
    c = lax.axis_index("c")
    s = lax.axis_index("s")
    bb = s // 4
    q = s % 4
    b = c * 4 + bb

    def zero(i, _):
        acc[pl.ds(i * L, L)] = jnp.zeros((L,), jnp.float32)
        return 0
    lax.fori_loop(0, NN // L, zero, 0)

    pltpu.sync_copy(r16_h.at[pl.ds(b * L, L)], rb)
    iot = lax.iota(jnp.int32, L)
    rsp = rb[...]
    bsp = jnp.full((L,), b * NN, jnp.int32)

    def process_chunk(base):
        pltpu.sync_copy(hflat.at[hix.at[pl.ds(base, C)]], hv)
        for j in range(C // L):
            t16 = ctl[pl.ds(base + j * L, L)]
            val = cw[pl.ds(base + j * L, L)] * hv[pl.ds(j * L, L)]
            chk = plsc.load_gather(acc, [t16])

            def rmw(ck):
                need = ck < val
                plsc.store_scatter(acc, [t16], jnp.maximum(ck, val),
                                   mask=need)
                return plsc.load_gather(acc, [t16])

            lax.while_loop(lambda ck: jnp.any(ck < val), rmw, chk)

    srcs = (heads_h, tails_h, types_h, wts_h)
    bufs = (hb, tb, yb, wb)

    def fire(k, boffset, sm):
        off = q * EQ + k * BLK
        for src, buf in zip(srcs, bufs):
            pltpu.async_copy(src.at[pl.ds(off, BLK)],
                             buf.at[pl.ds(boffset, BLK)], sm)

    def drain(sm):
        for src, buf in zip(srcs, bufs):
            pltpu.make_async_copy(src.at[pl.ds(0, BLK)],
                                  buf.at[pl.ds(0, BLK)], sm).wait()

    fire(0, 0, sem0)

    def block(k, fill):
        sl = k % 2
        o2 = sl * BLK

        @pl.when((k + 1 < NBLK) & (sl == 0))
        def _():
            fire(k + 1, BLK, sem1)

        @pl.when((k + 1 < NBLK) & (sl == 1))
        def _():
            fire(k + 1, 0, sem0)

        @pl.when(sl == 0)
        def _():
            drain(sem0)

        @pl.when(sl == 1)
        def _():
            drain(sem1)

        def one(base, fl):
            m = yb[pl.ds(base, L)] == rsp
            csum = plsc.cumsum(m.astype(jnp.int32))
            pos = jnp.full((L,), fl, jnp.int32) + csum - 1
            plsc.store_scatter(hix, [pos], hb[pl.ds(base, L)] + bsp, mask=m)
            plsc.store_scatter(ctl, [pos], tb[pl.ds(base, L)], mask=m)
            plsc.store_scatter(cw, [pos], wb[pl.ds(base, L)], mask=m)
            return fl + plsc.all_reduce_population_count(m)[0]

        def grp(g, fl):
            base = o2 + g * (2 * L)
            return one(base + L, one(base, fl))

        fill = lax.fori_loop(0, BLK // (2 * L), grp, fill)

        nch = fill // C

        def ch(j, _):
            process_chunk(j * C)
            return 0
        lax.fori_loop(0, nch, ch, 0)

        rem = fill - nch * C

        def mv(i, _):
            so = nch * C + i * L
            v = hix[pl.ds(so, L)]
            hix[pl.ds(i * L, L)] = v
            v = ctl[pl.ds(so, L)]
            ctl[pl.ds(i * L, L)] = v
            v = cw[pl.ds(so, L)]
            cw[pl.ds(i * L, L)] = v
            return 0
        lax.fori_loop(0, (rem + L - 1) // L, mv, 0)
        return rem

    fill = lax.fori_loop(0, NBLK, block, jnp.int32(0))

    fsp = jnp.full((L,), fill, jnp.int32)
    for i in range(C // L):
        lane = jnp.full((L,), i * L, jnp.int32) + iot
        pad = lane >= fsp
        hix[pl.ds(i * L, L)] = jnp.where(pad, bsp, hix[pl.ds(i * L, L)])
        ctl[pl.ds(i * L, L)] = jnp.where(pad, 0, ctl[pl.ds(i * L, L)])
        cw[pl.ds(i * L, L)] = jnp.where(pad, 0.0, cw[pl.ds(i * L, L)])
    process_chunk(0)

    pltpu.sync_copy(acc, part_h.at[pl.ds((b * 4 + q) * NN, NN)])
    plsc.subcore_barrier()

    node0 = q * NQ
    for st, sz in [(i * BLK, BLK) for i in range(15)] + [(15 * BLK, 1000)]:
        for c4 in range(4):
            pltpu.sync_copy(
                part_h.at[pl.ds((b * 4 + c4) * NN + node0 + st, sz)],
                cb.at[pl.ds(c4 * BLK, sz)])

        def cmb(i, _):
            o = i * L
            v = jnp.maximum(
                jnp.maximum(cb[pl.ds(o, L)], cb[pl.ds(BLK + o, L)]),
                jnp.maximum(cb[pl.ds(2 * BLK + o, L)],
                            cb[pl.ds(3 * BLK + o, L)]))
            ob[pl.ds(o, L)] = v
            return 0
        lax.fori_loop(0, sz // L, cmb, 0)
        if sz % L:
            o = sz - L
            v = jnp.maximum(
                jnp.maximum(cb[pl.ds(o, L)], cb[pl.ds(BLK + o, L)]),
                jnp.maximum(cb[pl.ds(2 * BLK + o, L)],
                            cb[pl.ds(3 * BLK + o, L)]))
            ob[pl.ds(o, L)] = v
        pltpu.sync_copy(ob.at[pl.ds(0, sz)],
                        out_h.at[pl.ds(b * NN + node0 + st, sz)])


def kernel(h_prob, edge_weight, edge_index, edge_type, r_index):
    hflat = h_prob.reshape(-1)
    heads = edge_index[0]
    tails = edge_index[1]
    r16 = jnp.repeat(r_index.astype(jnp.int32), L)
    out, _ = _sc_traverse(hflat, heads, tails, edge_type, edge_weight, r16)
    return out.reshape(NB, NN)
